# baseline (speedup 1.0000x reference)
import functools

import jax
import jax.numpy as jnp
from jax import lax
from jax.experimental import pallas as pl
from jax.experimental.pallas import tpu as pltpu
from jax.experimental.pallas import tpu_sc as plsc

HIDDEN = 256
NUM_ROWS = 100000
CHUNK = 400
NCHUNKS = NUM_ROWS // CHUNK
NC, NS = 2, 16
NW = NC * NS
ITERS = -(-NCHUNKS // NW)

_mesh = plsc.VectorSubcoreMesh(core_axis_name="c", subcore_axis_name="s")


@functools.partial(
    pl.kernel,
    out_type=jax.ShapeDtypeStruct((NUM_ROWS, HIDDEN), jnp.float32),
    mesh=_mesh,
    scratch_types=[
        pltpu.VMEM((CHUNK,), jnp.int32),
        pltpu.VMEM((CHUNK, HIDDEN), jnp.float32),
        pltpu.SemaphoreType.DMA,
    ],
)
def _emb_lookup(x_hbm, tab_hbm, out_hbm, idx_v, rows_v, sem):
    wid = lax.axis_index("s") * NC + lax.axis_index("c")
    subs = ((0, 96), (96, 96), (192, 96), (288, 112))

    def body(i, carry):
        chunk = wid + i * NW

        @pl.when(chunk < NCHUNKS)
        def _():
            base = chunk * CHUNK
            pltpu.sync_copy(x_hbm.at[pl.ds(base, CHUNK)], idx_v)
            descs = [
                pltpu.async_copy(
                    tab_hbm.at[idx_v.at[pl.ds(off, n)]],
                    rows_v.at[pl.ds(off, n)], sem)
                for off, n in subs
            ]
            for d in descs:
                d.wait()
            pltpu.sync_copy(rows_v, out_hbm.at[pl.ds(base, CHUNK)])

        return carry

    lax.fori_loop(0, ITERS, body, 0)


def kernel(x, emb_weight):
    return _emb_lookup(x.astype(jnp.int32), emb_weight)


# speedup vs baseline: 2.1118x; 2.1118x over previous
import functools

import jax
import jax.numpy as jnp
from jax import lax
from jax.experimental import pallas as pl
from jax.experimental.pallas import tpu as pltpu
from jax.experimental.pallas import tpu_sc as plsc

HIDDEN = 256
NUM_ROWS = 100000
CHUNK = 400
NCHUNKS = NUM_ROWS // CHUNK
NC, NS = 2, 16
NW = NC * NS
ITERS = -(-NCHUNKS // NW)

_mesh = plsc.VectorSubcoreMesh(core_axis_name="c", subcore_axis_name="s")


@functools.partial(
    pl.kernel,
    out_type=jax.ShapeDtypeStruct((NUM_ROWS, HIDDEN), jnp.float32),
    mesh=_mesh,
    scratch_types=[
        pltpu.VMEM((CHUNK,), jnp.int32),
        pltpu.VMEM((CHUNK, HIDDEN), jnp.float32),
        pltpu.SemaphoreType.DMA,
    ],
)
def _emb_lookup(x_hbm, tab_hbm, out_hbm, idx_v, rows_v, sem):
    wid = lax.axis_index("s") * NC + lax.axis_index("c")
    subs = ((0, 96), (96, 96), (192, 96), (288, 112))

    def body(i, carry):
        chunk = wid + i * NW

        @pl.when(chunk < NCHUNKS)
        def _():
            base = chunk * CHUNK
            pltpu.sync_copy(x_hbm.at[pl.ds(base, CHUNK)], idx_v)
            pltpu.async_copy(tab_hbm.at[idx_v], rows_v, sem).wait()
            pltpu.sync_copy(rows_v, out_hbm.at[pl.ds(base, CHUNK)])

        return carry

    lax.fori_loop(0, ITERS, body, 0)


def kernel(x, emb_weight):
    copy_id = (jnp.arange(NUM_ROWS, dtype=jnp.int32) // CHUNK) % NW
    x_adj = x.astype(jnp.int32) + 95 * copy_id
    tab_rep = jnp.tile(emb_weight, (NW, 1))
    return _emb_lookup(x_adj, tab_rep)


# speedup vs baseline: 2.1274x; 1.0074x over previous
import functools

import jax
import jax.numpy as jnp
from jax import lax
from jax.experimental import pallas as pl
from jax.experimental.pallas import tpu as pltpu
from jax.experimental.pallas import tpu_sc as plsc

HIDDEN = 256
NUM_EMB_ROWS = 95
NUM_ROWS = 100000
CHUNK = 200
NCHUNKS = NUM_ROWS // CHUNK
NC, NS = 2, 16
NW = NC * NS
ITERS_W = -(-NCHUNKS // NW)

_mesh = plsc.VectorSubcoreMesh(core_axis_name="c", subcore_axis_name="s")


@functools.partial(
    pl.kernel,
    out_type=jax.ShapeDtypeStruct((NUM_ROWS, HIDDEN), jnp.float32),
    mesh=_mesh,
    scratch_types=[
        pltpu.VMEM((CHUNK,), jnp.int32),
        pltpu.VMEM((CHUNK,), jnp.int32),
        pltpu.VMEM((CHUNK, HIDDEN), jnp.float32),
        pltpu.VMEM((CHUNK, HIDDEN), jnp.float32),
        pltpu.SemaphoreType.DMA,
        pltpu.SemaphoreType.DMA,
        pltpu.SemaphoreType.DMA,
        pltpu.SemaphoreType.DMA,
    ],
)
def _emb_lookup(x_hbm, tab_hbm, out_hbm, idx0, idx1, rows0, rows1,
                g0, g1, s0, s1):
    wid = lax.axis_index("s") * NC + lax.axis_index("c")
    idx = (idx0, idx1)
    rows = (rows0, rows1)
    gsem = (g0, g1)
    ssem = (s0, s1)

    def start_gather(j):
        b = j & 1
        base = (wid + j * NW) * CHUNK
        pltpu.sync_copy(x_hbm.at[pl.ds(base, CHUNK)], idx[b])
        return pltpu.async_copy(tab_hbm.at[idx[b]], rows[b], gsem[b])

    def start_store(j):
        b = j & 1
        base = (wid + j * NW) * CHUNK
        return pltpu.async_copy(rows[b], out_hbm.at[pl.ds(base, CHUNK)], ssem[b])

    gd = [None] * ITERS_W
    sd = [None] * ITERS_W
    for j in range(ITERS_W - 1):
        if j >= 2:
            sd[j - 2].wait()
        gd[j] = start_gather(j)
        if j >= 1:
            gd[j - 1].wait()
            sd[j - 1] = start_store(j - 1)

    last = ITERS_W - 1
    gd[last - 1].wait()
    sd[last - 1] = start_store(last - 1)
    sd[last - 2].wait()

    @pl.when(wid + last * NW < NCHUNKS)
    def _():
        start_gather(last).wait()
        start_store(last).wait()

    sd[last - 1].wait()


def kernel(x, emb_weight):
    copy_id = (jnp.arange(NUM_ROWS, dtype=jnp.int32) // CHUNK) % NW
    x_adj = x.astype(jnp.int32) + NUM_EMB_ROWS * copy_id
    tab_rep = jnp.tile(emb_weight, (NW, 1))
    return _emb_lookup(x_adj, tab_rep)
